# baseline (speedup 1.0000x reference)
import functools
import math

import jax
import jax.numpy as jnp
from jax import lax
from jax.experimental import pallas as pl
from jax.experimental.pallas import tpu as pltpu
from jax.experimental.pallas import tpu_sc as plsc

D_MODEL = 128
LANES = 16
NUM_WORKERS = 32
CHUNK = 128


@functools.partial(jax.jit, static_argnums=(2,))
def _emb_lookup(idx_flat, table, n_total):
    n_per_w = n_total // NUM_WORKERS
    n_chunks = n_per_w // CHUNK
    scale = jnp.float32(math.sqrt(D_MODEL))
    mesh = plsc.VectorSubcoreMesh(core_axis_name="c", subcore_axis_name="s")

    @functools.partial(
        pl.kernel,
        mesh=mesh,
        out_type=jax.ShapeDtypeStruct((n_total, D_MODEL), jnp.float32),
        scratch_types=[
            pltpu.VMEM((CHUNK,), jnp.int32),
            pltpu.VMEM((CHUNK, D_MODEL), jnp.float32),
            pltpu.SemaphoreType.DMA,
        ],
    )
    def k(idx_hbm, table_hbm, out_hbm, idx_v, rows_v, sem):
        wid = lax.axis_index("s") * 2 + lax.axis_index("c")
        base = wid * n_per_w

        def chunk_body(c, carry):
            off = base + c * CHUNK
            pltpu.sync_copy(idx_hbm.at[pl.ds(off, CHUNK)], idx_v)
            pltpu.async_copy(table_hbm.at[idx_v], rows_v, sem).wait()

            def row_body(r, carry2):
                for j in range(D_MODEL // LANES):
                    sl = pl.ds(j * LANES, LANES)
                    rows_v[r, sl] = rows_v[r, sl] * scale
                return carry2

            lax.fori_loop(0, CHUNK, row_body, 0)
            pltpu.sync_copy(rows_v, out_hbm.at[pl.ds(off, CHUNK)])
            return carry

        lax.fori_loop(0, n_chunks, chunk_body, 0)

    return k(idx_flat, table)


def kernel(x, table):
    n_total = x.shape[0] * x.shape[1]
    out = _emb_lookup(x.reshape(n_total), table, n_total)
    return out.reshape(x.shape[0], x.shape[1], table.shape[1])


# speedup vs baseline: 1.8645x; 1.8645x over previous
import functools
import math

import jax
import jax.numpy as jnp
from jax import lax
from jax.experimental import pallas as pl
from jax.experimental.pallas import tpu as pltpu
from jax.experimental.pallas import tpu_sc as plsc

D_MODEL = 128
LANES = 16
NUM_WORKERS = 32
CHUNK = 128
NBUF = 2


@functools.partial(jax.jit, static_argnums=(2,))
def _emb_lookup(idx2d, table, n_total):
    n_per_w = n_total // NUM_WORKERS
    n_chunks = n_per_w // CHUNK
    n_outer = n_chunks // NBUF
    scale = jnp.float32(math.sqrt(D_MODEL))
    mesh = plsc.VectorSubcoreMesh(core_axis_name="c", subcore_axis_name="s")

    @functools.partial(
        pl.kernel,
        mesh=mesh,
        out_type=jax.ShapeDtypeStruct((n_total, D_MODEL), jnp.float32),
        scratch_types=[
            pltpu.VMEM((n_chunks, CHUNK), jnp.int32),
            pltpu.VMEM((NBUF, CHUNK, D_MODEL), jnp.float32),
            pltpu.VMEM((NBUF, CHUNK, D_MODEL), jnp.float32),
        ]
        + [pltpu.SemaphoreType.DMA] * (2 * NBUF),
    )
    def k(idx_hbm, table_hbm, out_hbm, idx_v, rows_g, rows_s, *sems):
        sem_g = sems[:NBUF]
        sem_s = sems[NBUF:]
        wid = lax.axis_index("s") * 2 + lax.axis_index("c")
        base = wid * n_per_w

        pltpu.sync_copy(idx_hbm.at[wid], idx_v)

        def gather_start(c, b):
            pltpu.async_copy(table_hbm.at[idx_v.at[c]], rows_g.at[b], sem_g[b])

        def gather_wait(c, b):
            pltpu.make_async_copy(
                table_hbm.at[idx_v.at[c]], rows_g.at[b], sem_g[b]
            ).wait()

        def store_start(c, b):
            pltpu.async_copy(
                rows_s.at[b], out_hbm.at[pl.ds(base + c * CHUNK, CHUNK)], sem_s[b]
            )

        def store_wait(c, b):
            pltpu.make_async_copy(
                rows_s.at[b], out_hbm.at[pl.ds(base + c * CHUNK, CHUNK)], sem_s[b]
            ).wait()

        for b in range(NBUF):
            gather_start(b, b)

        def outer_body(o, carry):
            for b in range(NBUF):
                c = o * NBUF + b
                gather_wait(c, b)

                @pl.when(o > 0)
                def _():
                    store_wait(c - NBUF, b)

                g_b = rows_g.at[b]
                s_b = rows_s.at[b]

                @plsc.parallel_loop(0, CHUNK, unroll=2)
                def _(r):
                    for j in range(D_MODEL // LANES):
                        sl = pl.ds(j * LANES, LANES)
                        s_b[r, sl] = g_b[r, sl] * scale

                store_start(c, b)

                @pl.when(o < n_outer - 1)
                def _():
                    gather_start(c + NBUF, b)
            return carry

        lax.fori_loop(0, n_outer, outer_body, 0)

        for b in range(NBUF):
            store_wait((n_outer - 1) * NBUF + b, b)

    return k(idx2d, table)


def kernel(x, table):
    n_total = x.shape[0] * x.shape[1]
    idx2d = x.reshape(NUM_WORKERS, n_total // (NUM_WORKERS * CHUNK), CHUNK)
    out = _emb_lookup(idx2d, table, n_total)
    return out.reshape(x.shape[0], x.shape[1], table.shape[1])
